# Initial kernel scaffold; baseline (speedup 1.0000x reference)
import jax
import jax.numpy as jnp
from jax.experimental import pallas as pl


def kernel(logits, x_mask):
    raise NotImplementedError("write your pallas kernel here")


# speedup vs baseline: 187.2203x; 187.2203x over previous
import functools

import jax
import jax.numpy as jnp
from jax.experimental import pallas as pl

_B = 32
_N = 32


def _suffix_excl(d):
    t = d
    n = d.shape[0]
    sh = 1
    while sh < n:
        t = t + jnp.concatenate(
            [t[sh:], jnp.zeros((sh,) + t.shape[1:], t.dtype)], axis=0
        )
        sh *= 2
    return t - d


def _stick_body(lt_ref, xt_ref, out_ref):
    def row(m, A):
        lrow = jnp.squeeze(lt_ref[pl.ds(m, 1)], axis=0)
        xrow = jnp.squeeze(xt_ref[pl.ds(m, 1)], axis=0)
        brow = jax.nn.sigmoid(lrow)
        c2 = xrow * brow
        c1 = xrow - c2
        D = jnp.maximum(xrow - A, 0.0)
        S = _suffix_excl(D)
        U = 1.0 - A
        t = jnp.ones((1, _B), jnp.float32)
        ps = []
        for n in range(_N):
            p = c1[n : n + 1] * jnp.maximum(t - S[n : n + 1], 0.0) + c2[
                n : n + 1
            ] * jnp.minimum(t, U[n : n + 1])
            ps.append(p)
            t = t - p
        p_row = jnp.concatenate(ps, axis=0)
        out_ref[pl.ds(m, 1)] = p_row[None]
        return A + p_row

    jax.lax.fori_loop(0, _N, row, jnp.zeros((_N, _B), jnp.float32))


@functools.partial(jax.jit, static_argnames=())
def kernel(logits, x_mask):
    lt = jnp.transpose(logits, (1, 2, 0))
    xt = jnp.transpose(x_mask, (1, 2, 0))
    out_t = pl.pallas_call(
        _stick_body,
        out_shape=jax.ShapeDtypeStruct((_N, _N, _B), jnp.float32),
    )(lt, xt)
    return jnp.transpose(out_t, (2, 0, 1))
